# initial kernel scaffold (unmeasured)
import jax
import jax.numpy as jnp
from jax import lax
from jax.experimental import pallas as pl
from jax.experimental.pallas import tpu as pltpu

M = 4096
D = 4096
F_SHARD = 8192
MHALF = 2048
Q = 1024

BN = 2048
BK = 512

MESH = pl.DeviceIdType.MESH


def _matmul_body(a_ref, b_ref, o_ref, acc_ref):
    k = pl.program_id(1)

    @pl.when(k == 0)
    def _():
        acc_ref[...] = jnp.zeros_like(acc_ref)

    a = a_ref[...].astype(jnp.bfloat16)
    b = b_ref[...].astype(jnp.bfloat16)
    acc_ref[...] += lax.dot_general(
        a, b, (((1,), (1,)), ((), ())), preferred_element_type=jnp.float32
    )

    @pl.when(k == pl.num_programs(1) - 1)
    def _():
        o_ref[...] = acc_ref[...].astype(jnp.bfloat16)


def _partial_gemm(dy_half, w):
    return pl.pallas_call(
        _matmul_body,
        grid=(D // BN, F_SHARD // BK),
        in_specs=[
            pl.BlockSpec((MHALF, BK), lambda n, k: (0, k)),
            pl.BlockSpec((BN, BK), lambda n, k: (n, k)),
        ],
        out_specs=pl.BlockSpec((MHALF, BN), lambda n, k: (0, n)),
        out_shape=jax.ShapeDtypeStruct((MHALF, D), jnp.bfloat16),
        scratch_shapes=[pltpu.VMEM((MHALF, BN), jnp.float32)],
    )(dy_half, w)


def _ar_body(p_ref, out_ref, r_y1, r_x, r_y2, r_diag,
             send_sems, recv_sems, copy_sems):
    my_x = lax.axis_index("x")
    my_y = lax.axis_index("y")
    nbr_y = (my_x, 1 - my_y)
    nbr_x = (1 - my_x, my_y)

    barrier = pltpu.get_barrier_semaphore()
    pl.semaphore_signal(barrier, 1, device_id=nbr_y, device_id_type=MESH)
    pl.semaphore_signal(barrier, 1, device_id=nbr_x, device_id_type=MESH)
    pl.semaphore_wait(barrier, 2)

    mine = pl.ds(my_y * Q, Q)
    theirs = pl.ds((1 - my_y) * Q, Q)

    rdma1 = pltpu.make_async_remote_copy(
        src_ref=p_ref.at[theirs],
        dst_ref=r_y1,
        send_sem=send_sems.at[0],
        recv_sem=recv_sems.at[0],
        device_id=nbr_y,
        device_id_type=MESH,
    )
    rdma1.start()
    rdma1.wait()

    q32 = p_ref[mine, :].astype(jnp.float32) + r_y1[...].astype(jnp.float32)
    p_ref[mine, :] = q32.astype(jnp.bfloat16)

    my_q = 2 * my_x + my_y
    c0 = pltpu.make_async_copy(
        p_ref.at[mine], out_ref.at[pl.ds(my_q * Q, Q)], copy_sems.at[0]
    )
    c0.start()

    rdma2x = pltpu.make_async_remote_copy(
        src_ref=p_ref.at[mine], dst_ref=r_x,
        send_sem=send_sems.at[1], recv_sem=recv_sems.at[1],
        device_id=nbr_x, device_id_type=MESH,
    )
    rdma2y = pltpu.make_async_remote_copy(
        src_ref=p_ref.at[mine], dst_ref=r_y2,
        send_sem=send_sems.at[2], recv_sem=recv_sems.at[2],
        device_id=nbr_y, device_id_type=MESH,
    )
    rdma2x.start()
    rdma2y.start()
    rdma2x.wait()
    rdma2y.wait()

    rdma3 = pltpu.make_async_remote_copy(
        src_ref=r_y2, dst_ref=r_diag,
        send_sem=send_sems.at[3], recv_sem=recv_sems.at[3],
        device_id=nbr_x, device_id_type=MESH,
    )
    rdma3.start()

    qx = 2 * (1 - my_x) + my_y
    c1 = pltpu.make_async_copy(
        r_x, out_ref.at[pl.ds(qx * Q, Q)], copy_sems.at[1]
    )
    c1.start()
    qy = 2 * my_x + (1 - my_y)
    c2 = pltpu.make_async_copy(
        r_y2, out_ref.at[pl.ds(qy * Q, Q)], copy_sems.at[2]
    )
    c2.start()

    rdma3.wait()
    qd = 2 * (1 - my_x) + (1 - my_y)
    c3 = pltpu.make_async_copy(
        r_diag, out_ref.at[pl.ds(qd * Q, Q)], copy_sems.at[3]
    )
    c3.start()

    c0.wait()
    c1.wait()
    c2.wait()
    c3.wait()


def _all_reduce(p):
    return pl.pallas_call(
        _ar_body,
        in_specs=[pl.BlockSpec(memory_space=pltpu.VMEM)],
        out_specs=pl.BlockSpec(memory_space=pltpu.ANY),
        out_shape=jax.ShapeDtypeStruct((M, D), jnp.bfloat16),
        scratch_shapes=[
            pltpu.VMEM((Q, D), jnp.bfloat16),
            pltpu.VMEM((Q, D), jnp.bfloat16),
            pltpu.VMEM((Q, D), jnp.bfloat16),
            pltpu.VMEM((Q, D), jnp.bfloat16),
            pltpu.SemaphoreType.DMA((4,)),
            pltpu.SemaphoreType.DMA((4,)),
            pltpu.SemaphoreType.DMA((4,)),
        ],
        compiler_params=pltpu.CompilerParams(collective_id=0),
    )(p)


def kernel(dy, W):
    my_x = lax.axis_index("x")
    dy_half = lax.dynamic_slice_in_dim(dy, my_x * MHALF, MHALF, axis=0)
    p = _partial_gemm(dy_half, W)
    return _all_reduce(p)


# baseline (device time: 519183 ns/iter reference)
import jax
import jax.numpy as jnp
from jax import lax
from jax.experimental import pallas as pl
from jax.experimental.pallas import tpu as pltpu

M = 4096
D = 4096
F_SHARD = 8192
MHALF = 2048
Q = 1024

BN = 2048
BK = 512

MESH = pl.DeviceIdType.MESH


def _matmul_body(a_ref, b_ref, o_ref, acc_ref):
    k = pl.program_id(1)

    @pl.when(k == 0)
    def _():
        acc_ref[...] = jnp.zeros_like(acc_ref)

    a = a_ref[...].astype(jnp.bfloat16)
    b = b_ref[...].astype(jnp.bfloat16)
    acc_ref[...] += lax.dot_general(
        a, b, (((1,), (1,)), ((), ())), preferred_element_type=jnp.float32
    )

    @pl.when(k == pl.num_programs(1) - 1)
    def _():
        o_ref[...] = acc_ref[...].astype(jnp.bfloat16)


def _partial_gemm(dy_half, w):
    return pl.pallas_call(
        _matmul_body,
        grid=(D // BN, F_SHARD // BK),
        in_specs=[
            pl.BlockSpec((MHALF, BK), lambda n, k: (0, k)),
            pl.BlockSpec((BN, BK), lambda n, k: (n, k)),
        ],
        out_specs=pl.BlockSpec((MHALF, BN), lambda n, k: (0, n)),
        out_shape=jax.ShapeDtypeStruct((MHALF, D), jnp.bfloat16),
        scratch_shapes=[pltpu.VMEM((MHALF, BN), jnp.float32)],
        compiler_params=pltpu.CompilerParams(
            vmem_limit_bytes=64 * 1024 * 1024
        ),
    )(dy_half, w)


def _ar_body(p_ref, out_ref, r_y1, r_x, r_y2, r_diag,
             send_sems, recv_sems, copy_sems):
    my_x = lax.axis_index("x")
    my_y = lax.axis_index("y")
    nbr_y = (my_x, 1 - my_y)
    nbr_x = (1 - my_x, my_y)

    barrier = pltpu.get_barrier_semaphore()
    pl.semaphore_signal(barrier, 1, device_id=nbr_y, device_id_type=MESH)
    pl.semaphore_signal(barrier, 1, device_id=nbr_x, device_id_type=MESH)
    pl.semaphore_wait(barrier, 2)

    mine = pl.ds(my_y * Q, Q)
    theirs = pl.ds((1 - my_y) * Q, Q)

    rdma1 = pltpu.make_async_remote_copy(
        src_ref=p_ref.at[theirs],
        dst_ref=r_y1,
        send_sem=send_sems.at[0],
        recv_sem=recv_sems.at[0],
        device_id=nbr_y,
        device_id_type=MESH,
    )
    rdma1.start()
    rdma1.wait()

    q32 = p_ref[mine, :].astype(jnp.float32) + r_y1[...].astype(jnp.float32)
    p_ref[mine, :] = q32.astype(jnp.bfloat16)

    my_q = 2 * my_x + my_y
    c0 = pltpu.make_async_copy(
        p_ref.at[mine], out_ref.at[pl.ds(my_q * Q, Q)], copy_sems.at[0]
    )
    c0.start()

    rdma2x = pltpu.make_async_remote_copy(
        src_ref=p_ref.at[mine], dst_ref=r_x,
        send_sem=send_sems.at[1], recv_sem=recv_sems.at[1],
        device_id=nbr_x, device_id_type=MESH,
    )
    rdma2y = pltpu.make_async_remote_copy(
        src_ref=p_ref.at[mine], dst_ref=r_y2,
        send_sem=send_sems.at[2], recv_sem=recv_sems.at[2],
        device_id=nbr_y, device_id_type=MESH,
    )
    rdma2x.start()
    rdma2y.start()
    rdma2x.wait()
    rdma2y.wait()

    rdma3 = pltpu.make_async_remote_copy(
        src_ref=r_y2, dst_ref=r_diag,
        send_sem=send_sems.at[3], recv_sem=recv_sems.at[3],
        device_id=nbr_x, device_id_type=MESH,
    )
    rdma3.start()

    qx = 2 * (1 - my_x) + my_y
    c1 = pltpu.make_async_copy(
        r_x, out_ref.at[pl.ds(qx * Q, Q)], copy_sems.at[1]
    )
    c1.start()
    qy = 2 * my_x + (1 - my_y)
    c2 = pltpu.make_async_copy(
        r_y2, out_ref.at[pl.ds(qy * Q, Q)], copy_sems.at[2]
    )
    c2.start()

    rdma3.wait()
    qd = 2 * (1 - my_x) + (1 - my_y)
    c3 = pltpu.make_async_copy(
        r_diag, out_ref.at[pl.ds(qd * Q, Q)], copy_sems.at[3]
    )
    c3.start()

    c0.wait()
    c1.wait()
    c2.wait()
    c3.wait()


def _all_reduce(p):
    return pl.pallas_call(
        _ar_body,
        in_specs=[pl.BlockSpec(memory_space=pltpu.VMEM)],
        out_specs=pl.BlockSpec(memory_space=pl.ANY),
        out_shape=jax.ShapeDtypeStruct((M, D), jnp.bfloat16),
        scratch_shapes=[
            pltpu.VMEM((Q, D), jnp.bfloat16),
            pltpu.VMEM((Q, D), jnp.bfloat16),
            pltpu.VMEM((Q, D), jnp.bfloat16),
            pltpu.VMEM((Q, D), jnp.bfloat16),
            pltpu.SemaphoreType.DMA((4,)),
            pltpu.SemaphoreType.DMA((4,)),
            pltpu.SemaphoreType.DMA((4,)),
        ],
        compiler_params=pltpu.CompilerParams(
            collective_id=0, vmem_limit_bytes=64 * 1024 * 1024
        ),
    )(p)


def kernel(dy, W):
    my_x = lax.axis_index("x")
    dy_half = lax.dynamic_slice_in_dim(dy, my_x * MHALF, MHALF, axis=0)
    p = _partial_gemm(dy_half, W)
    return _all_reduce(p)


# device time: 403419 ns/iter; 1.2870x vs baseline; 1.2870x over previous
import jax
import jax.numpy as jnp
from jax import lax
from jax.experimental import pallas as pl
from jax.experimental.pallas import tpu as pltpu

M = 4096
D = 4096
F_SHARD = 8192
MHALF = 2048
Q = 1024

BN = 2048
BK = 512

NC = 4
CH = Q // NC

MESH = pl.DeviceIdType.MESH


def _matmul_body(s_ref, a_ref, b_ref, o_ref, acc_ref):
    k = pl.program_id(1)

    @pl.when(k == 0)
    def _():
        acc_ref[...] = jnp.zeros_like(acc_ref)

    a = a_ref[...].astype(jnp.bfloat16)
    b = b_ref[...].astype(jnp.bfloat16)
    acc_ref[...] += lax.dot_general(
        a, b, (((1,), (1,)), ((), ())), preferred_element_type=jnp.float32
    )

    @pl.when(k == pl.num_programs(1) - 1)
    def _():
        o_ref[...] = acc_ref[...].astype(jnp.bfloat16)


def _partial_gemm(dy, w, my_x):
    grid_spec = pltpu.PrefetchScalarGridSpec(
        num_scalar_prefetch=1,
        grid=(D // BN, F_SHARD // BK),
        in_specs=[
            pl.BlockSpec((MHALF, BK), lambda n, k, s: (s[0], k)),
            pl.BlockSpec((BN, BK), lambda n, k, s: (n, k)),
        ],
        out_specs=pl.BlockSpec((MHALF, BN), lambda n, k, s: (0, n)),
        scratch_shapes=[pltpu.VMEM((MHALF, BN), jnp.float32)],
    )
    return pl.pallas_call(
        _matmul_body,
        grid_spec=grid_spec,
        out_shape=jax.ShapeDtypeStruct((MHALF, D), jnp.bfloat16),
        compiler_params=pltpu.CompilerParams(
            vmem_limit_bytes=64 * 1024 * 1024
        ),
    )(jnp.reshape(my_x, (1,)).astype(jnp.int32), dy, w)


def _ar_body(p_ref, out_ref, r_y1, r_x, r_y2, r_diag,
             s1s, s1r, s2xs, s2xr, s2ys, s2yr, s3s, s3r, copy_sems):
    my_x = lax.axis_index("x")
    my_y = lax.axis_index("y")
    nbr_y = (my_x, 1 - my_y)
    nbr_x = (1 - my_x, my_y)

    barrier = pltpu.get_barrier_semaphore()
    pl.semaphore_signal(barrier, 1, device_id=nbr_y, device_id_type=MESH)
    pl.semaphore_signal(barrier, 1, device_id=nbr_x, device_id_type=MESH)
    pl.semaphore_wait(barrier, 2)

    mine0 = my_y * Q
    theirs0 = (1 - my_y) * Q

    my_q = 2 * my_x + my_y
    qx = 2 * (1 - my_x) + my_y
    qy = 2 * my_x + (1 - my_y)
    qd = 2 * (1 - my_x) + (1 - my_y)

    rdma1 = []
    for c in range(NC):
        r = pltpu.make_async_remote_copy(
            src_ref=p_ref.at[pl.ds(theirs0 + c * CH, CH)],
            dst_ref=r_y1.at[pl.ds(c * CH, CH)],
            send_sem=s1s.at[c], recv_sem=s1r.at[c],
            device_id=nbr_y, device_id_type=MESH,
        )
        r.start()
        rdma1.append(r)

    rdma2x, rdma2y, copies = [], [], []
    for c in range(NC):
        rdma1[c].wait()
        rows = pl.ds(mine0 + c * CH, CH)
        q32 = (p_ref[rows, :].astype(jnp.float32)
               + r_y1[pl.ds(c * CH, CH), :].astype(jnp.float32))
        p_ref[rows, :] = q32.astype(jnp.bfloat16)

        cp = pltpu.make_async_copy(
            p_ref.at[rows],
            out_ref.at[pl.ds(my_q * Q + c * CH, CH)],
            copy_sems.at[c],
        )
        cp.start()
        copies.append(cp)

        r2x = pltpu.make_async_remote_copy(
            src_ref=p_ref.at[rows],
            dst_ref=r_x.at[pl.ds(c * CH, CH)],
            send_sem=s2xs.at[c], recv_sem=s2xr.at[c],
            device_id=nbr_x, device_id_type=MESH,
        )
        r2x.start()
        rdma2x.append(r2x)
        r2y = pltpu.make_async_remote_copy(
            src_ref=p_ref.at[rows],
            dst_ref=r_y2.at[pl.ds(c * CH, CH)],
            send_sem=s2ys.at[c], recv_sem=s2yr.at[c],
            device_id=nbr_y, device_id_type=MESH,
        )
        r2y.start()
        rdma2y.append(r2y)

    rdma3 = []
    for c in range(NC):
        rdma2y[c].wait()
        chunk = pl.ds(c * CH, CH)
        r3 = pltpu.make_async_remote_copy(
            src_ref=r_y2.at[chunk],
            dst_ref=r_diag.at[chunk],
            send_sem=s3s.at[c], recv_sem=s3r.at[c],
            device_id=nbr_x, device_id_type=MESH,
        )
        r3.start()
        rdma3.append(r3)
        cp = pltpu.make_async_copy(
            r_y2.at[chunk],
            out_ref.at[pl.ds(qy * Q + c * CH, CH)],
            copy_sems.at[NC + c],
        )
        cp.start()
        copies.append(cp)

    for c in range(NC):
        rdma2x[c].wait()
        chunk = pl.ds(c * CH, CH)
        cp = pltpu.make_async_copy(
            r_x.at[chunk],
            out_ref.at[pl.ds(qx * Q + c * CH, CH)],
            copy_sems.at[2 * NC + c],
        )
        cp.start()
        copies.append(cp)

    for c in range(NC):
        rdma3[c].wait()
        chunk = pl.ds(c * CH, CH)
        cp = pltpu.make_async_copy(
            r_diag.at[chunk],
            out_ref.at[pl.ds(qd * Q + c * CH, CH)],
            copy_sems.at[3 * NC + c],
        )
        cp.start()
        copies.append(cp)

    for cp in copies:
        cp.wait()


def _all_reduce(p):
    return pl.pallas_call(
        _ar_body,
        in_specs=[pl.BlockSpec(memory_space=pltpu.VMEM)],
        out_specs=pl.BlockSpec(memory_space=pl.ANY),
        out_shape=jax.ShapeDtypeStruct((M, D), jnp.bfloat16),
        scratch_shapes=[
            pltpu.VMEM((Q, D), jnp.bfloat16),
            pltpu.VMEM((Q, D), jnp.bfloat16),
            pltpu.VMEM((Q, D), jnp.bfloat16),
            pltpu.VMEM((Q, D), jnp.bfloat16),
            pltpu.SemaphoreType.DMA((NC,)),
            pltpu.SemaphoreType.DMA((NC,)),
            pltpu.SemaphoreType.DMA((NC,)),
            pltpu.SemaphoreType.DMA((NC,)),
            pltpu.SemaphoreType.DMA((NC,)),
            pltpu.SemaphoreType.DMA((NC,)),
            pltpu.SemaphoreType.DMA((NC,)),
            pltpu.SemaphoreType.DMA((NC,)),
            pltpu.SemaphoreType.DMA((4 * NC,)),
        ],
        compiler_params=pltpu.CompilerParams(
            collective_id=0, vmem_limit_bytes=64 * 1024 * 1024
        ),
    )(p)


def kernel(dy, W):
    my_x = lax.axis_index("x")
    p = _partial_gemm(dy, W, my_x)
    return _all_reduce(p)


# device time: 356846 ns/iter; 1.4549x vs baseline; 1.1305x over previous
import jax
import jax.numpy as jnp
from jax import lax
from jax.experimental import pallas as pl
from jax.experimental.pallas import tpu as pltpu

M = 4096
D = 4096
F_SHARD = 8192
Q = 1024

BN = 1024
BK = 512
NT = D // BN
KS = F_SHARD // BK

MESH = pl.DeviceIdType.MESH


def _body(s_ref, a_ref, b_ref, out_ref, acc_ref, p_th, m_buf, r_y1,
          rs_s, rs_r, agx_s, agx_r, agy_s, agy_r, fwd_s, fwd_r, cp_sems):
    n = pl.program_id(0)
    m = pl.program_id(1)
    k = pl.program_id(2)

    my_x = lax.axis_index("x")
    my_y = lax.axis_index("y")
    nbr_y = (my_x, 1 - my_y)
    nbr_x = (1 - my_x, my_y)
    my_q = 2 * my_x + my_y
    qy = 2 * my_x + (1 - my_y)

    @pl.when(jnp.logical_and(n == 0, jnp.logical_and(m == 0, k == 0)))
    def _():
        barrier = pltpu.get_barrier_semaphore()
        pl.semaphore_signal(barrier, 1, device_id=nbr_y, device_id_type=MESH)
        pl.semaphore_signal(barrier, 1, device_id=nbr_x, device_id_type=MESH)
        pl.semaphore_wait(barrier, 2)

    @pl.when(k == 0)
    def _():
        acc_ref[...] = jnp.zeros_like(acc_ref)

    a = a_ref[...].astype(jnp.bfloat16)
    b = b_ref[...].astype(jnp.bfloat16)
    acc_ref[...] += lax.dot_general(
        a, b, (((1,), (1,)), ((), ())), preferred_element_type=jnp.float32
    )

    def cols(c):
        return pl.ds(c * BN, BN)

    def make_rs(c):
        return pltpu.make_async_remote_copy(
            src_ref=p_th.at[:, cols(c)], dst_ref=r_y1.at[:, cols(c)],
            send_sem=rs_s.at[c], recv_sem=rs_r.at[c],
            device_id=nbr_y, device_id_type=MESH,
        )

    def make_agx(c):
        return pltpu.make_async_remote_copy(
            src_ref=m_buf.at[:, cols(c)],
            dst_ref=out_ref.at[pl.ds(my_q * Q, Q), cols(c)],
            send_sem=agx_s.at[c], recv_sem=agx_r.at[c],
            device_id=nbr_x, device_id_type=MESH,
        )

    def make_agy(c):
        return pltpu.make_async_remote_copy(
            src_ref=m_buf.at[:, cols(c)],
            dst_ref=out_ref.at[pl.ds(my_q * Q, Q), cols(c)],
            send_sem=agy_s.at[c], recv_sem=agy_r.at[c],
            device_id=nbr_y, device_id_type=MESH,
        )

    def make_fwd(c):
        region = (pl.ds(qy * Q, Q), cols(c))
        return pltpu.make_async_remote_copy(
            src_ref=out_ref.at[region[0], region[1]],
            dst_ref=out_ref.at[region[0], region[1]],
            send_sem=fwd_s.at[c], recv_sem=fwd_r.at[c],
            device_id=nbr_x, device_id_type=MESH,
        )

    def make_out_copy(c):
        return pltpu.make_async_copy(
            m_buf.at[:, cols(c)],
            out_ref.at[pl.ds(my_q * Q, Q), cols(c)],
            cp_sems.at[c],
        )

    def process_add(c):
        make_rs(c).wait()
        s32 = (m_buf[:, cols(c)].astype(jnp.float32)
               + r_y1[:, cols(c)].astype(jnp.float32))
        m_buf[:, cols(c)] = s32.astype(jnp.bfloat16)
        make_out_copy(c).start()
        make_agx(c).start()
        make_agy(c).start()

    def process_fwd(c):
        make_agy(c).wait()
        make_fwd(c).start()

    for i in range(NT):
        @pl.when(jnp.logical_and(n == i, jnp.logical_and(m == 0, k == KS - 1)))
        def _(i=i):
            p_th[:, cols(i)] = acc_ref[...].astype(jnp.bfloat16)
            make_rs(i).start()
            if i >= 1:
                process_add(i - 1)
            if i >= 2:
                process_fwd(i - 2)

        @pl.when(jnp.logical_and(n == i, jnp.logical_and(m == 1, k == KS - 1)))
        def _(i=i):
            m_buf[:, cols(i)] = acc_ref[...].astype(jnp.bfloat16)
            if i == NT - 1:
                process_add(NT - 1)
                process_fwd(NT - 2)
                process_fwd(NT - 1)
                for c in range(NT):
                    make_fwd(c).wait()
                for c in range(NT):
                    make_agx(c).wait()
                for c in range(NT):
                    make_out_copy(c).wait()


def kernel(dy, W):
    my_x = lax.axis_index("x")
    my_y = lax.axis_index("y")
    sref = jnp.stack([my_x, my_y]).astype(jnp.int32)

    grid_spec = pltpu.PrefetchScalarGridSpec(
        num_scalar_prefetch=1,
        grid=(NT, 2, KS),
        in_specs=[
            pl.BlockSpec(
                (Q, BK),
                lambda n, m, k, s: (2 * s[0] + (1 - m) * (1 - s[1]) + m * s[1], k),
            ),
            pl.BlockSpec((BN, BK), lambda n, m, k, s: (n, k)),
        ],
        out_specs=pl.BlockSpec(memory_space=pl.ANY),
        scratch_shapes=[
            pltpu.VMEM((Q, BN), jnp.float32),
            pltpu.VMEM((Q, D), jnp.bfloat16),
            pltpu.VMEM((Q, D), jnp.bfloat16),
            pltpu.VMEM((Q, D), jnp.bfloat16),
            pltpu.SemaphoreType.DMA((NT,)),
            pltpu.SemaphoreType.DMA((NT,)),
            pltpu.SemaphoreType.DMA((NT,)),
            pltpu.SemaphoreType.DMA((NT,)),
            pltpu.SemaphoreType.DMA((NT,)),
            pltpu.SemaphoreType.DMA((NT,)),
            pltpu.SemaphoreType.DMA((NT,)),
            pltpu.SemaphoreType.DMA((NT,)),
            pltpu.SemaphoreType.DMA((NT,)),
        ],
    )
    return pl.pallas_call(
        _body,
        grid_spec=grid_spec,
        out_shape=jax.ShapeDtypeStruct((M, D), jnp.bfloat16),
        compiler_params=pltpu.CompilerParams(
            collective_id=0, vmem_limit_bytes=64 * 1024 * 1024
        ),
    )(sref, dy, W)


# device time: 343817 ns/iter; 1.5101x vs baseline; 1.0379x over previous
import jax
import jax.numpy as jnp
from jax import lax
from jax.experimental import pallas as pl
from jax.experimental.pallas import tpu as pltpu

M = 4096
D = 4096
F_SHARD = 8192
Q = 1024

BN = 1024
BK = 1024
NT = D // BN
KS = F_SHARD // BK

MESH = pl.DeviceIdType.MESH


def _body(s_ref, a_ref, b_ref, out_ref, acc0, acc1, p_th, m_buf, r_y1,
          rs_s, rs_r, agx_s, agx_r, agy_s, agy_r, fwd_s, fwd_r, cp_sems):
    n = pl.program_id(0)
    k = pl.program_id(1)
    m = pl.program_id(2)

    my_x = lax.axis_index("x")
    my_y = lax.axis_index("y")
    nbr_y = (my_x, 1 - my_y)
    nbr_x = (1 - my_x, my_y)
    my_q = 2 * my_x + my_y
    qy = 2 * my_x + (1 - my_y)

    @pl.when(jnp.logical_and(n == 0, jnp.logical_and(k == 0, m == 0)))
    def _():
        barrier = pltpu.get_barrier_semaphore()
        pl.semaphore_signal(barrier, 1, device_id=nbr_y, device_id_type=MESH)
        pl.semaphore_signal(barrier, 1, device_id=nbr_x, device_id_type=MESH)
        pl.semaphore_wait(barrier, 2)

    a = a_ref[...].astype(jnp.bfloat16)
    b = b_ref[...].astype(jnp.bfloat16)
    prod = lax.dot_general(
        a, b, (((1,), (1,)), ((), ())), preferred_element_type=jnp.float32
    )

    @pl.when(m == 0)
    def _():
        @pl.when(k == 0)
        def _():
            acc0[...] = jnp.zeros_like(acc0)
        acc0[...] += prod

    @pl.when(m == 1)
    def _():
        @pl.when(k == 0)
        def _():
            acc1[...] = jnp.zeros_like(acc1)
        acc1[...] += prod

    def cols(c):
        return pl.ds(c * BN, BN)

    def make_rs(c):
        return pltpu.make_async_remote_copy(
            src_ref=p_th.at[:, cols(c)], dst_ref=r_y1.at[:, cols(c)],
            send_sem=rs_s.at[c], recv_sem=rs_r.at[c],
            device_id=nbr_y, device_id_type=MESH,
        )

    def make_agx(c):
        return pltpu.make_async_remote_copy(
            src_ref=m_buf.at[:, cols(c)],
            dst_ref=out_ref.at[pl.ds(my_q * Q, Q), cols(c)],
            send_sem=agx_s.at[c], recv_sem=agx_r.at[c],
            device_id=nbr_x, device_id_type=MESH,
        )

    def make_agy(c):
        return pltpu.make_async_remote_copy(
            src_ref=m_buf.at[:, cols(c)],
            dst_ref=out_ref.at[pl.ds(my_q * Q, Q), cols(c)],
            send_sem=agy_s.at[c], recv_sem=agy_r.at[c],
            device_id=nbr_y, device_id_type=MESH,
        )

    def make_fwd(c):
        return pltpu.make_async_remote_copy(
            src_ref=out_ref.at[pl.ds(qy * Q, Q), cols(c)],
            dst_ref=out_ref.at[pl.ds(qy * Q, Q), cols(c)],
            send_sem=fwd_s.at[c], recv_sem=fwd_r.at[c],
            device_id=nbr_x, device_id_type=MESH,
        )

    def make_out_copy(c):
        return pltpu.make_async_copy(
            m_buf.at[:, cols(c)],
            out_ref.at[pl.ds(my_q * Q, Q), cols(c)],
            cp_sems.at[c],
        )

    def process_add(c):
        make_rs(c).wait()
        s32 = (m_buf[:, cols(c)].astype(jnp.float32)
               + r_y1[:, cols(c)].astype(jnp.float32))
        m_buf[:, cols(c)] = s32.astype(jnp.bfloat16)
        make_out_copy(c).start()
        make_agx(c).start()
        make_agy(c).start()

    def process_fwd(c):
        make_agy(c).wait()
        make_fwd(c).start()

    for i in range(NT):
        if i >= 2:
            @pl.when(jnp.logical_and(n == i,
                                     jnp.logical_and(k == KS - 4, m == 0)))
            def _(i=i):
                process_fwd(i - 2)

        if i >= 1:
            @pl.when(jnp.logical_and(n == i,
                                     jnp.logical_and(k == KS - 3, m == 0)))
            def _(i=i):
                process_add(i - 1)

        @pl.when(jnp.logical_and(n == i,
                                 jnp.logical_and(k == KS - 1, m == 0)))
        def _(i=i):
            p_th[:, cols(i)] = acc0[...].astype(jnp.bfloat16)
            make_rs(i).start()

        @pl.when(jnp.logical_and(n == i,
                                 jnp.logical_and(k == KS - 1, m == 1)))
        def _(i=i):
            m_buf[:, cols(i)] = acc1[...].astype(jnp.bfloat16)
            if i == NT - 1:
                process_add(NT - 1)
                process_fwd(NT - 2)
                process_fwd(NT - 1)
                for c in range(NT):
                    make_fwd(c).wait()
                for c in range(NT):
                    make_agx(c).wait()
                for c in range(NT):
                    make_out_copy(c).wait()


def kernel(dy, W):
    my_x = lax.axis_index("x")
    my_y = lax.axis_index("y")
    sref = jnp.stack([my_x, my_y]).astype(jnp.int32)

    grid_spec = pltpu.PrefetchScalarGridSpec(
        num_scalar_prefetch=1,
        grid=(NT, KS, 2),
        in_specs=[
            pl.BlockSpec(
                (Q, BK),
                lambda n, k, m, s: (2 * s[0] + (1 - m) * (1 - s[1]) + m * s[1], k),
            ),
            pl.BlockSpec((BN, BK), lambda n, k, m, s: (n, k)),
        ],
        out_specs=pl.BlockSpec(memory_space=pl.ANY),
        scratch_shapes=[
            pltpu.VMEM((Q, BN), jnp.float32),
            pltpu.VMEM((Q, BN), jnp.float32),
            pltpu.VMEM((Q, D), jnp.bfloat16),
            pltpu.VMEM((Q, D), jnp.bfloat16),
            pltpu.VMEM((Q, D), jnp.bfloat16),
            pltpu.SemaphoreType.DMA((NT,)),
            pltpu.SemaphoreType.DMA((NT,)),
            pltpu.SemaphoreType.DMA((NT,)),
            pltpu.SemaphoreType.DMA((NT,)),
            pltpu.SemaphoreType.DMA((NT,)),
            pltpu.SemaphoreType.DMA((NT,)),
            pltpu.SemaphoreType.DMA((NT,)),
            pltpu.SemaphoreType.DMA((NT,)),
            pltpu.SemaphoreType.DMA((NT,)),
        ],
    )
    return pl.pallas_call(
        _body,
        grid_spec=grid_spec,
        out_shape=jax.ShapeDtypeStruct((M, D), jnp.bfloat16),
        compiler_params=pltpu.CompilerParams(
            collective_id=0, vmem_limit_bytes=64 * 1024 * 1024
        ),
    )(sref, dy, W)


# device time: 327869 ns/iter; 1.5835x vs baseline; 1.0486x over previous
import jax
import jax.numpy as jnp
from jax import lax
from jax.experimental import pallas as pl
from jax.experimental.pallas import tpu as pltpu

M = 4096
D = 4096
F_SHARD = 8192
Q = 1024

BN = 1024
BK = 1024
NT = D // BN
KS = F_SHARD // BK

CW = 512
NCH = D // CW

MESH = pl.DeviceIdType.MESH


def _body(s_ref, a_ref, b_ref, out_ref, acc0, acc1, p_th, m_buf, r_y1,
          rs_s, rs_r, agx_s, agx_r, agy_s, agy_r, fwd_s, fwd_r, cp_sems):
    n = pl.program_id(0)
    k = pl.program_id(1)
    m = pl.program_id(2)

    my_x = lax.axis_index("x")
    my_y = lax.axis_index("y")
    nbr_y = (my_x, 1 - my_y)
    nbr_x = (1 - my_x, my_y)
    my_q = 2 * my_x + my_y
    qy = 2 * my_x + (1 - my_y)

    @pl.when(jnp.logical_and(n == 0, jnp.logical_and(k == 0, m == 0)))
    def _():
        barrier = pltpu.get_barrier_semaphore()
        pl.semaphore_signal(barrier, 1, device_id=nbr_y, device_id_type=MESH)
        pl.semaphore_signal(barrier, 1, device_id=nbr_x, device_id_type=MESH)
        pl.semaphore_wait(barrier, 2)

    a = a_ref[...].astype(jnp.bfloat16)
    b = b_ref[...].astype(jnp.bfloat16)
    prod = lax.dot_general(
        a, b, (((1,), (1,)), ((), ())), preferred_element_type=jnp.float32
    )

    @pl.when(m == 0)
    def _():
        @pl.when(k == 0)
        def _():
            acc0[...] = jnp.zeros_like(acc0)
        acc0[...] += prod

    @pl.when(m == 1)
    def _():
        @pl.when(k == 0)
        def _():
            acc1[...] = jnp.zeros_like(acc1)
        acc1[...] += prod

    def cols(c):
        return pl.ds(c * CW, CW)

    def make_rs(c):
        return pltpu.make_async_remote_copy(
            src_ref=p_th.at[:, cols(c)], dst_ref=r_y1.at[:, cols(c)],
            send_sem=rs_s.at[c], recv_sem=rs_r.at[c],
            device_id=nbr_y, device_id_type=MESH,
        )

    def make_agx(c):
        return pltpu.make_async_remote_copy(
            src_ref=m_buf.at[:, cols(c)],
            dst_ref=out_ref.at[pl.ds(my_q * Q, Q), cols(c)],
            send_sem=agx_s.at[c], recv_sem=agx_r.at[c],
            device_id=nbr_x, device_id_type=MESH,
        )

    def make_agy(c):
        return pltpu.make_async_remote_copy(
            src_ref=m_buf.at[:, cols(c)],
            dst_ref=out_ref.at[pl.ds(my_q * Q, Q), cols(c)],
            send_sem=agy_s.at[c], recv_sem=agy_r.at[c],
            device_id=nbr_y, device_id_type=MESH,
        )

    def make_fwd(c):
        return pltpu.make_async_remote_copy(
            src_ref=out_ref.at[pl.ds(qy * Q, Q), cols(c)],
            dst_ref=out_ref.at[pl.ds(qy * Q, Q), cols(c)],
            send_sem=fwd_s.at[c], recv_sem=fwd_r.at[c],
            device_id=nbr_x, device_id_type=MESH,
        )

    def make_out_copy(c):
        return pltpu.make_async_copy(
            m_buf.at[:, cols(c)],
            out_ref.at[pl.ds(my_q * Q, Q), cols(c)],
            cp_sems.at[c],
        )

    def process_add(c):
        make_rs(c).wait()
        s32 = (m_buf[:, cols(c)].astype(jnp.float32)
               + r_y1[:, cols(c)].astype(jnp.float32))
        m_buf[:, cols(c)] = s32.astype(jnp.bfloat16)
        make_out_copy(c).start()
        make_agx(c).start()
        make_agy(c).start()

    def process_fwd(c):
        make_agy(c).wait()
        make_fwd(c).start()

    def at_step(i, kk, mm):
        return jnp.logical_and(n == i, jnp.logical_and(k == kk, m == mm))

    for i in range(NT):
        if i >= 2:
            @pl.when(at_step(i, KS - 6, 0))
            def _(i=i):
                process_fwd(2 * (i - 2))

            @pl.when(at_step(i, KS - 5, 0))
            def _(i=i):
                process_fwd(2 * (i - 2) + 1)

        if i >= 1:
            @pl.when(at_step(i, KS - 4, 0))
            def _(i=i):
                process_add(2 * (i - 1))

            @pl.when(at_step(i, KS - 3, 0))
            def _(i=i):
                process_add(2 * (i - 1) + 1)

        @pl.when(at_step(i, KS - 1, 0))
        def _(i=i):
            p_th[:, pl.ds(i * BN, BN)] = acc0[...].astype(jnp.bfloat16)
            make_rs(2 * i).start()
            make_rs(2 * i + 1).start()

        @pl.when(at_step(i, KS - 1, 1))
        def _(i=i):
            m_buf[:, pl.ds(i * BN, BN)] = acc1[...].astype(jnp.bfloat16)
            if i == NT - 1:
                process_add(NCH - 2)
                process_add(NCH - 1)
                for c in (2 * (NT - 2), 2 * (NT - 2) + 1, NCH - 2, NCH - 1):
                    process_fwd(c)
                for c in range(NCH):
                    make_fwd(c).wait()
                for c in range(NCH):
                    make_agx(c).wait()
                for c in range(NCH):
                    make_out_copy(c).wait()


def kernel(dy, W):
    my_x = lax.axis_index("x")
    my_y = lax.axis_index("y")
    sref = jnp.stack([my_x, my_y]).astype(jnp.int32)

    grid_spec = pltpu.PrefetchScalarGridSpec(
        num_scalar_prefetch=1,
        grid=(NT, KS, 2),
        in_specs=[
            pl.BlockSpec(
                (Q, BK),
                lambda n, k, m, s: (2 * s[0] + (1 - m) * (1 - s[1]) + m * s[1], k),
            ),
            pl.BlockSpec((BN, BK), lambda n, k, m, s: (n, k)),
        ],
        out_specs=pl.BlockSpec(memory_space=pl.ANY),
        scratch_shapes=[
            pltpu.VMEM((Q, BN), jnp.float32),
            pltpu.VMEM((Q, BN), jnp.float32),
            pltpu.VMEM((Q, D), jnp.bfloat16),
            pltpu.VMEM((Q, D), jnp.bfloat16),
            pltpu.VMEM((Q, D), jnp.bfloat16),
            pltpu.SemaphoreType.DMA((NCH,)),
            pltpu.SemaphoreType.DMA((NCH,)),
            pltpu.SemaphoreType.DMA((NCH,)),
            pltpu.SemaphoreType.DMA((NCH,)),
            pltpu.SemaphoreType.DMA((NCH,)),
            pltpu.SemaphoreType.DMA((NCH,)),
            pltpu.SemaphoreType.DMA((NCH,)),
            pltpu.SemaphoreType.DMA((NCH,)),
            pltpu.SemaphoreType.DMA((NCH,)),
        ],
    )
    return pl.pallas_call(
        _body,
        grid_spec=grid_spec,
        out_shape=jax.ShapeDtypeStruct((M, D), jnp.bfloat16),
        compiler_params=pltpu.CompilerParams(
            collective_id=0, vmem_limit_bytes=64 * 1024 * 1024
        ),
    )(sref, dy, W)


# device time: 324522 ns/iter; 1.5998x vs baseline; 1.0103x over previous
import jax
import jax.numpy as jnp
from jax import lax
from jax.experimental import pallas as pl
from jax.experimental.pallas import tpu as pltpu

M = 4096
D = 4096
F_SHARD = 8192
Q = 1024

BN = 1024
BK = 1024
NT = D // BN
KS = F_SHARD // BK

CHUNKS = [(0, 512), (512, 512), (1024, 512), (1536, 512), (2048, 512),
          (2560, 512), (3072, 256), (3328, 256), (3584, 256), (3840, 256)]
TILE_CHUNKS = {0: [0, 1], 1: [2, 3], 2: [4, 5], 3: [6, 7, 8, 9]}
NCH = len(CHUNKS)

MESH = pl.DeviceIdType.MESH


def _body(s_ref, a_ref, b_ref, out_ref, acc0, acc1, p_th, m_buf, r_y1,
          rs_s, rs_r, agx_s, agx_r, agy_s, agy_r, fwd_s, fwd_r, cp_sems):
    n = pl.program_id(0)
    k = pl.program_id(1)
    m = pl.program_id(2)

    my_x = lax.axis_index("x")
    my_y = lax.axis_index("y")
    nbr_y = (my_x, 1 - my_y)
    nbr_x = (1 - my_x, my_y)
    my_q = 2 * my_x + my_y
    qy = 2 * my_x + (1 - my_y)

    @pl.when(jnp.logical_and(n == 0, jnp.logical_and(k == 0, m == 0)))
    def _():
        barrier = pltpu.get_barrier_semaphore()
        pl.semaphore_signal(barrier, 1, device_id=nbr_y, device_id_type=MESH)
        pl.semaphore_signal(barrier, 1, device_id=nbr_x, device_id_type=MESH)
        pl.semaphore_wait(barrier, 2)

    a = a_ref[...].astype(jnp.bfloat16)
    b = b_ref[...].astype(jnp.bfloat16)
    prod = lax.dot_general(
        a, b, (((1,), (1,)), ((), ())), preferred_element_type=jnp.float32
    )

    @pl.when(m == 0)
    def _():
        @pl.when(k == 0)
        def _():
            acc0[...] = jnp.zeros_like(acc0)
        acc0[...] += prod

    @pl.when(m == 1)
    def _():
        @pl.when(k == 0)
        def _():
            acc1[...] = jnp.zeros_like(acc1)
        acc1[...] += prod

    def cols(c):
        off, w = CHUNKS[c]
        return pl.ds(off, w)

    def make_rs(c):
        return pltpu.make_async_remote_copy(
            src_ref=p_th.at[:, cols(c)], dst_ref=r_y1.at[:, cols(c)],
            send_sem=rs_s.at[c], recv_sem=rs_r.at[c],
            device_id=nbr_y, device_id_type=MESH,
        )

    def make_agx(c):
        return pltpu.make_async_remote_copy(
            src_ref=m_buf.at[:, cols(c)],
            dst_ref=out_ref.at[pl.ds(my_q * Q, Q), cols(c)],
            send_sem=agx_s.at[c], recv_sem=agx_r.at[c],
            device_id=nbr_x, device_id_type=MESH,
        )

    def make_agy(c):
        return pltpu.make_async_remote_copy(
            src_ref=m_buf.at[:, cols(c)],
            dst_ref=out_ref.at[pl.ds(my_q * Q, Q), cols(c)],
            send_sem=agy_s.at[c], recv_sem=agy_r.at[c],
            device_id=nbr_y, device_id_type=MESH,
        )

    def make_fwd(c):
        return pltpu.make_async_remote_copy(
            src_ref=out_ref.at[pl.ds(qy * Q, Q), cols(c)],
            dst_ref=out_ref.at[pl.ds(qy * Q, Q), cols(c)],
            send_sem=fwd_s.at[c], recv_sem=fwd_r.at[c],
            device_id=nbr_x, device_id_type=MESH,
        )

    def make_out_copy(c):
        return pltpu.make_async_copy(
            m_buf.at[:, cols(c)],
            out_ref.at[pl.ds(my_q * Q, Q), cols(c)],
            cp_sems.at[c],
        )

    def process_add(c):
        make_rs(c).wait()
        s32 = (m_buf[:, cols(c)].astype(jnp.float32)
               + r_y1[:, cols(c)].astype(jnp.float32))
        m_buf[:, cols(c)] = s32.astype(jnp.bfloat16)
        make_out_copy(c).start()
        make_agx(c).start()
        make_agy(c).start()

    def process_fwd(c):
        make_agy(c).wait()
        make_fwd(c).start()

    def at_step(i, kk, mm):
        return jnp.logical_and(n == i, jnp.logical_and(k == kk, m == mm))

    for i in range(NT):
        if i >= 2:
            for j, c in enumerate(TILE_CHUNKS[i - 2]):
                @pl.when(at_step(i, KS - 6 + j, 0))
                def _(c=c):
                    process_fwd(c)

        if i >= 1:
            for j, c in enumerate(TILE_CHUNKS[i - 1]):
                @pl.when(at_step(i, KS - 4 + j, 0))
                def _(c=c):
                    process_add(c)

        @pl.when(at_step(i, KS - 1, 0))
        def _(i=i):
            p_th[:, pl.ds(i * BN, BN)] = acc0[...].astype(jnp.bfloat16)
            for c in TILE_CHUNKS[i]:
                make_rs(c).start()

        @pl.when(at_step(i, KS - 1, 1))
        def _(i=i):
            m_buf[:, pl.ds(i * BN, BN)] = acc1[...].astype(jnp.bfloat16)
            if i == NT - 1:
                for c in TILE_CHUNKS[NT - 1]:
                    process_add(c)
                for c in TILE_CHUNKS[NT - 2] + TILE_CHUNKS[NT - 1]:
                    process_fwd(c)
                for c in range(NCH):
                    make_fwd(c).wait()
                for c in range(NCH):
                    make_agx(c).wait()
                for c in range(NCH):
                    make_out_copy(c).wait()


def kernel(dy, W):
    my_x = lax.axis_index("x")
    my_y = lax.axis_index("y")
    sref = jnp.stack([my_x, my_y]).astype(jnp.int32)

    grid_spec = pltpu.PrefetchScalarGridSpec(
        num_scalar_prefetch=1,
        grid=(NT, KS, 2),
        in_specs=[
            pl.BlockSpec(
                (Q, BK),
                lambda n, k, m, s: (2 * s[0] + (1 - m) * (1 - s[1]) + m * s[1], k),
            ),
            pl.BlockSpec((BN, BK), lambda n, k, m, s: (n, k)),
        ],
        out_specs=pl.BlockSpec(memory_space=pl.ANY),
        scratch_shapes=[
            pltpu.VMEM((Q, BN), jnp.float32),
            pltpu.VMEM((Q, BN), jnp.float32),
            pltpu.VMEM((Q, D), jnp.bfloat16),
            pltpu.VMEM((Q, D), jnp.bfloat16),
            pltpu.VMEM((Q, D), jnp.bfloat16),
            pltpu.SemaphoreType.DMA((NCH,)),
            pltpu.SemaphoreType.DMA((NCH,)),
            pltpu.SemaphoreType.DMA((NCH,)),
            pltpu.SemaphoreType.DMA((NCH,)),
            pltpu.SemaphoreType.DMA((NCH,)),
            pltpu.SemaphoreType.DMA((NCH,)),
            pltpu.SemaphoreType.DMA((NCH,)),
            pltpu.SemaphoreType.DMA((NCH,)),
            pltpu.SemaphoreType.DMA((NCH,)),
        ],
    )
    return pl.pallas_call(
        _body,
        grid_spec=grid_spec,
        out_shape=jax.ShapeDtypeStruct((M, D), jnp.bfloat16),
        compiler_params=pltpu.CompilerParams(
            collective_id=0, vmem_limit_bytes=64 * 1024 * 1024
        ),
    )(sref, dy, W)
